# baseline (speedup 1.0000x reference)
import functools

import jax
import jax.numpy as jnp
from jax import lax
from jax.experimental import pallas as pl
from jax.experimental.pallas import tpu as pltpu
from jax.experimental.pallas import tpu_sc as plsc

B, L, A = 128, 512, 1024
B_SC = 32
B_TC = B - B_SC
BB = 8
CH = 32
RU = 4
JU = 8
NLANE = 16


def _tc_matvec_kernel(d_ref, w_ref, b_ref, o_ref):
    d = d_ref[...]
    w = w_ref[:, :, :]
    acc = jnp.sum(d * w, axis=-1)
    o_ref[...] = acc[:, None, :] + b_ref[...]


def _tc_matvec(dict, w3, bias3):
    grid = (B_TC // BB,)
    out = pl.pallas_call(
        _tc_matvec_kernel,
        grid=grid,
        in_specs=[
            pl.BlockSpec((BB, L, A), lambda i: (i, 0, 0)),
            pl.BlockSpec((BB, 1, A), lambda i: (i, 0, 0)),
            pl.BlockSpec((BB, 1, 1), lambda i: (i, 0, 0)),
        ],
        out_specs=pl.BlockSpec((BB, 1, L), lambda i: (i, 0, 0)),
        out_shape=jax.ShapeDtypeStruct((B_TC, 1, L), jnp.float32),
    )(dict, w3, bias3)
    return out.reshape(B_TC, L)


def _sc_partials(dict, w):
    mesh = plsc.VectorSubcoreMesh(core_axis_name="c", subcore_axis_name="s")

    @functools.partial(
        pl.kernel,
        mesh=mesh,
        out_type=jax.ShapeDtypeStruct((B_SC, L, NLANE), jnp.float32),
        scratch_types=[
            pltpu.VMEM((A,), jnp.float32),
            pltpu.VMEM((CH, A), jnp.float32),
            pltpu.VMEM((CH, A), jnp.float32),
            pltpu.VMEM((CH, NLANE), jnp.float32),
            pltpu.VMEM((CH, NLANE), jnp.float32),
            pltpu.SemaphoreType.DMA,
            pltpu.SemaphoreType.DMA,
            pltpu.SemaphoreType.DMA,
            pltpu.SemaphoreType.DMA,
            pltpu.SemaphoreType.DMA,
        ],
    )
    def sc_k(d_hbm, w_hbm, out_hbm, w_v, buf0, buf1, part0, part1,
             sem0, sem1, semp0, semp1, semw):
        c = lax.axis_index("c")
        s = lax.axis_index("s")
        wid = s * 2 + c
        b = B_TC + wid

        pltpu.async_copy(w_hbm.at[b], w_v, semw).wait()

        n_chunks = L // CH
        bufs = (buf0, buf1)
        sems = (sem0, sem1)
        parts = (part0, part1)
        psems = (semp0, semp1)

        pltpu.async_copy(d_hbm.at[b, pl.ds(0, CH), :], buf0, sem0)

        def run_chunk(buf, part):
            def row_group(i, carry):
                r = i * RU

                def j_block(jj, accs):
                    accs = list(accs)
                    for dj in range(JU):
                        off = (jj * JU + dj) * NLANE
                        wj = w_v[pl.ds(off, NLANE)]
                        for k in range(RU):
                            accs[k] = accs[k] + buf[r + k, pl.ds(off, NLANE)] * wj
                    return tuple(accs)

                accs = lax.fori_loop(
                    0, A // (NLANE * JU), j_block,
                    tuple(jnp.zeros((NLANE,), jnp.float32) for _ in range(RU)),
                )
                for k in range(RU):
                    part[r + k, :] = accs[k]
                return carry

            lax.fori_loop(0, CH // RU, row_group, 0)

        for ci in range(n_chunks):
            pltpu.make_async_copy(
                d_hbm.at[b, pl.ds(ci * CH, CH), :], bufs[ci % 2], sems[ci % 2]
            ).wait()
            if ci + 1 < n_chunks:
                pltpu.async_copy(
                    d_hbm.at[b, pl.ds((ci + 1) * CH, CH), :],
                    bufs[(ci + 1) % 2],
                    sems[(ci + 1) % 2],
                )
            if ci >= 2:
                pltpu.make_async_copy(
                    parts[ci % 2],
                    out_hbm.at[wid, pl.ds((ci - 2) * CH, CH), :],
                    psems[ci % 2],
                ).wait()
            run_chunk(bufs[ci % 2], parts[ci % 2])
            pltpu.async_copy(
                parts[ci % 2],
                out_hbm.at[wid, pl.ds(ci * CH, CH), :],
                psems[ci % 2],
            )
        for ci in (n_chunks - 2, n_chunks - 1):
            pltpu.make_async_copy(
                parts[ci % 2],
                out_hbm.at[wid, pl.ds(ci * CH, CH), :],
                psems[ci % 2],
            ).wait()

    return sc_k(dict, w)


def _tc_finish_kernel(p_ref, b_ref, o_ref):
    acc = jnp.sum(p_ref[...], axis=-1)
    o_ref[...] = acc[:, None, :] + b_ref[...]


def _tc_finish(partials, bias3_sc):
    out = pl.pallas_call(
        _tc_finish_kernel,
        in_specs=[
            pl.BlockSpec((B_SC, L, NLANE), lambda: (0, 0, 0)),
            pl.BlockSpec((B_SC, 1, 1), lambda: (0, 0, 0)),
        ],
        out_specs=pl.BlockSpec((B_SC, 1, L), lambda: (0, 0, 0)),
        out_shape=jax.ShapeDtypeStruct((B_SC, 1, L), jnp.float32),
    )(partials, bias3_sc)
    return out.reshape(B_SC, L)


def kernel(dict, coef):
    w = coef[:, :A]
    w3 = coef[:, None, :A]
    bias3 = coef[:, None, A:]

    partials = _sc_partials(dict, w)
    out_tc = _tc_matvec(dict, w3[:B_TC], bias3[:B_TC])
    out_sc = _tc_finish(partials, bias3[B_TC:])
    return jnp.concatenate([out_tc, out_sc], axis=0)[:, :, None]


# speedup vs baseline: 1.1562x; 1.1562x over previous
import functools

import jax
import jax.numpy as jnp
from jax import lax
from jax.experimental import pallas as pl
from jax.experimental.pallas import tpu as pltpu
from jax.experimental.pallas import tpu_sc as plsc

B, L, A = 128, 512, 1024
AP = 1032
B_SC = 32
B_TC = B - B_SC
BB = 8
CH = 32
NBUF = 3
RU = 4
JU = 8
NLANE = 16


def _tc_matvec_kernel(d_ref, c_ref, o_ref):
    d = d_ref[...]
    w = c_ref[:, :, :A]
    bias = c_ref[:, :, A:A + 1]
    acc = jnp.sum(d * w, axis=-1)
    o_ref[...] = acc[:, None, :] + bias


def _tc_matvec(dict, coef_pad):
    grid = (B_TC // BB,)
    out = pl.pallas_call(
        _tc_matvec_kernel,
        grid=grid,
        in_specs=[
            pl.BlockSpec((BB, L, A), lambda i: (i, 0, 0)),
            pl.BlockSpec((BB, 1, AP), lambda i: (i, 0, 0)),
        ],
        out_specs=pl.BlockSpec((BB, 1, L), lambda i: (i, 0, 0)),
        out_shape=jax.ShapeDtypeStruct((B_TC, 1, L), jnp.float32),
    )(dict, coef_pad[:, None, :])
    return out.reshape(B_TC, L)


def _sc_matvec(dict, coef_pad):
    mesh = plsc.VectorSubcoreMesh(core_axis_name="c", subcore_axis_name="s")

    @functools.partial(
        pl.kernel,
        mesh=mesh,
        out_type=jax.ShapeDtypeStruct((B_SC, L), jnp.float32),
        scratch_types=[
            pltpu.VMEM((AP,), jnp.float32),
            pltpu.VMEM((NBUF, CH, A), jnp.float32),
            pltpu.VMEM((L,), jnp.float32),
            pltpu.SemaphoreType.DMA,
            pltpu.SemaphoreType.DMA,
            pltpu.SemaphoreType.DMA,
            pltpu.SemaphoreType.DMA,
        ],
    )
    def sc_k(d_hbm, c_hbm, out_hbm, w_v, ring, out_v, semw, s0, s1, s2):
        c = lax.axis_index("c")
        s = lax.axis_index("s")
        wid = s * 2 + c
        b = B_TC + wid

        pltpu.async_copy(c_hbm.at[b], w_v, semw).wait()
        bias = w_v[pl.ds(AP - NLANE, NLANE)][NLANE - (AP - A)]

        n_chunks = L // CH
        sems = (s0, s1, s2)

        def issue(ci):
            pltpu.async_copy(
                d_hbm.at[b, pl.ds(ci * CH, CH), :], ring.at[ci % NBUF], sems[ci % NBUF]
            )

        for ci in range(NBUF - 1):
            issue(ci)

        lane_i = lax.iota(jnp.int32, NLANE)

        def run_chunk(ci):
            buf = ring.at[ci % NBUF]

            def row_group(g, carry):
                def subgroup(sg, out_vec):
                    r = g * NLANE + sg * RU

                    def j_block(jj, accs):
                        accs = list(accs)
                        for dj in range(JU):
                            off = (jj * JU + dj) * NLANE
                            wj = w_v[pl.ds(off, NLANE)]
                            for k in range(RU):
                                accs[k] = accs[k] + buf[r + k, pl.ds(off, NLANE)] * wj
                        return tuple(accs)

                    accs = lax.fori_loop(
                        0, A // (NLANE * JU), j_block,
                        tuple(jnp.zeros((NLANE,), jnp.float32) for _ in range(RU)),
                    )
                    for k in range(RU):
                        v = accs[k]
                        for sh in (8, 4, 2, 1):
                            v = v + v.at[lane_i ^ sh].get(mode="promise_in_bounds")
                        out_vec = jnp.where(lane_i == sg * RU + k, v + bias, out_vec)
                    return out_vec

                out_vec = lax.fori_loop(
                    0, NLANE // RU, subgroup, jnp.zeros((NLANE,), jnp.float32)
                )
                out_v[pl.ds(ci * CH + g * NLANE, NLANE)] = out_vec
                return carry

            lax.fori_loop(0, CH // NLANE, row_group, 0)

        for ci in range(n_chunks):
            pltpu.make_async_copy(
                d_hbm.at[b, pl.ds(ci * CH, CH), :], ring.at[ci % NBUF], sems[ci % NBUF]
            ).wait()
            if ci + NBUF - 1 < n_chunks:
                issue(ci + NBUF - 1)
            run_chunk(ci)

        pltpu.sync_copy(out_v, out_hbm.at[wid])

    return sc_k(dict, coef_pad)


def kernel(dict, coef):
    coef_pad = jnp.pad(coef, ((0, 0), (0, AP - (A + 1))))
    out_sc = _sc_matvec(dict, coef_pad)
    out_tc = _tc_matvec(dict, coef_pad)
    return jnp.concatenate([out_tc, out_sc], axis=0)[:, :, None]
